# Initial kernel scaffold; baseline (speedup 1.0000x reference)
import jax
import jax.numpy as jnp
from jax.experimental import pallas as pl


def kernel(x, edge_index, edge_attr, node_types, node_emb, edge_emb, Win, bin_, Wl, bl, Wr, br, We, att, bg, Wout, bout):
    raise NotImplementedError("write your pallas kernel here")


# speedup vs baseline: 5.7089x; 5.7089x over previous
import functools

import jax
import jax.numpy as jnp
from jax.experimental import pallas as pl
from jax.experimental.pallas import tpu as pltpu

_N = 10000
_E = 160000
_HID = 64
_HEADS = 8
_LAYERS = 3
_NEG = 0.2
_F = _HEADS * _HID


def _mm_kernel(a_ref, b_ref, o_ref):
    o_ref[...] = jnp.dot(a_ref[...], b_ref[...],
                         preferred_element_type=jnp.float32)


def _mm(a, b, blk_rows):
    r, k = a.shape
    c = b.shape[1]
    return pl.pallas_call(
        _mm_kernel,
        grid=(r // blk_rows,),
        in_specs=[
            pl.BlockSpec((blk_rows, k), lambda i: (i, 0)),
            pl.BlockSpec((k, c), lambda i: (0, 0)),
        ],
        out_specs=pl.BlockSpec((blk_rows, c), lambda i: (i, 0)),
        out_shape=jax.ShapeDtypeStruct((r, c), jnp.float32),
    )(a, b)


def kernel(x, edge_index, edge_attr, node_types, node_emb, edge_emb,
           Win, bin_, Wl, bl, Wr, br, We, att, bg, Wout, bout):
    src = edge_index[0]
    dst = edge_index[1]

    nt_oh = jax.nn.one_hot(node_types, 5, dtype=jnp.float32)
    h = _mm(x, Win, 1000) + bin_ + nt_oh @ node_emb

    cnt = jnp.zeros((_N, 5), jnp.float32).at[dst, edge_attr].add(1.0)
    deg = cnt.sum(axis=1)
    emean = (cnt @ edge_emb) / jnp.maximum(deg, 1.0)[:, None]

    for l in range(_LAYERS):
        xl = _mm(h, Wl[l], 1000) + bl[l]
        xr = _mm(h, Wr[l], 1000) + br[l]
        ee_t = edge_emb @ We[l]
        ee_self = _mm(emean, We[l], 1000)

        xj = xl[src]
        u = xj + xr[dst] + ee_t[edge_attr]
        u = jnp.where(u >= 0, u, _NEG * u).reshape(_E, _HEADS, _HID)
        ea = jnp.exp((u * att[l][None]).sum(-1))
        msg = (xj.reshape(_E, _HEADS, _HID) * ea[:, :, None]).reshape(_E, _F)
        out_un = jnp.zeros((_N, _F), jnp.float32).at[dst].add(msg)
        den = jnp.zeros((_N, _HEADS), jnp.float32).at[dst].add(ea)

        us = xl + xr + ee_self
        us = jnp.where(us >= 0, us, _NEG * us).reshape(_N, _HEADS, _HID)
        eas = jnp.exp((us * att[l][None]).sum(-1))
        out_un = out_un + (xl.reshape(_N, _HEADS, _HID)
                           * eas[:, :, None]).reshape(_N, _F)
        den = den + eas

        out = out_un.reshape(_N, _HEADS, _HID) / (den + 1e-16)[:, :, None]
        h = jax.nn.relu(out.mean(axis=1) + bg[l])

    return _mm(h, Wout, 1000) + bout


# speedup vs baseline: 7.1292x; 1.2488x over previous
import functools

import jax
import jax.numpy as jnp
from jax import lax
from jax.experimental import pallas as pl
from jax.experimental.pallas import tpu as pltpu
from jax.experimental.pallas import tpu_sc as plsc

_N = 10000
_E = 160000
_HID = 64
_HEADS = 8
_LAYERS = 3
_NEG = 0.2
_F = _HEADS * _HID
_NSUB = 16
_EPW = _E // _NSUB
_CHUNK = 80
_NCHUNK = _EPW // _CHUNK
_NPAD = 10240
_NPW = _NPAD // _NSUB


def _mm_kernel(a_ref, b_ref, o_ref):
    o_ref[...] = jnp.dot(a_ref[...], b_ref[...],
                         preferred_element_type=jnp.float32)


def _mm(a, b, blk_rows):
    r, k = a.shape
    c = b.shape[1]
    return pl.pallas_call(
        _mm_kernel,
        grid=(r // blk_rows,),
        in_specs=[
            pl.BlockSpec((blk_rows, k), lambda i: (i, 0)),
            pl.BlockSpec((k, c), lambda i: (0, 0)),
        ],
        out_specs=pl.BlockSpec((blk_rows, c), lambda i: (i, 0)),
        out_shape=jax.ShapeDtypeStruct((r, c), jnp.float32),
    )(a, b)


_BE = 2000


def _edge_kernel(xj_ref, xrd_ref, oh_ref, eet_ref, attA_ref, expB_ref,
                 m0_ref, m1_ref, m2_ref, m3_ref, ea_ref):
    xj = xj_ref[...]
    u = xj + xrd_ref[...] + jnp.dot(oh_ref[...], eet_ref[...],
                                    preferred_element_type=jnp.float32)
    ul = jnp.maximum(u, _NEG * u)
    alpha = jnp.dot(ul, attA_ref[...], preferred_element_type=jnp.float32)
    ea = jnp.exp(alpha)
    m = xj * jnp.dot(ea, expB_ref[...],
                     preferred_element_type=jnp.float32)
    m0_ref[...] = m[:, 0:128]
    m1_ref[...] = m[:, 128:256]
    m2_ref[...] = m[:, 256:384]
    m3_ref[...] = m[:, 384:512]
    ea_ref[...] = ea


def _edge_math(xj, xrd, oh, ee_t8, attA, expB):
    grid = _E // _BE
    mspec = pl.BlockSpec((_BE, 128), lambda i: (i, 0))
    return pl.pallas_call(
        _edge_kernel,
        grid=(grid,),
        in_specs=[
            pl.BlockSpec((_BE, _F), lambda i: (i, 0)),
            pl.BlockSpec((_BE, _F), lambda i: (i, 0)),
            pl.BlockSpec((_BE, 8), lambda i: (i, 0)),
            pl.BlockSpec((8, _F), lambda i: (0, 0)),
            pl.BlockSpec((_F, 8), lambda i: (0, 0)),
            pl.BlockSpec((8, _F), lambda i: (0, 0)),
        ],
        out_specs=[mspec, mspec, mspec, mspec,
                   pl.BlockSpec((_BE, 8), lambda i: (i, 0))],
        out_shape=[jax.ShapeDtypeStruct((_E, 128), jnp.float32)] * 4
        + [jax.ShapeDtypeStruct((_E, 8), jnp.float32)],
    )(xj, xrd, oh, ee_t8, attA, expB)


def _sc_scatter_kernel(m0, m1, m2, m3, dst_hbm, ea_hbm, z128, z8,
                       o0, o1, o2, o3, den_hbm,
                       acc, den_acc, idx_v, rows_v, ea_v):
    c = lax.axis_index("c")
    s = lax.axis_index("s")
    msgs = (m0, m1, m2, m3)
    outs = (o0, o1, o2, o3)

    for g in range(2):
        pltpu.sync_copy(z128, acc.at[pl.ds(s * _NPW, _NPW)])
        if g == 0:
            @pl.when(c == 0)
            def _():
                pltpu.sync_copy(z8, den_acc.at[pl.ds(s * _NPW, _NPW)])

        plsc.subcore_barrier()

        for gc in range(2):
            @pl.when(c == gc)
            def _():
                mref = msgs[2 * gc + g]

                def body(i, carry):
                    base = s * _EPW + i * _CHUNK
                    pltpu.sync_copy(dst_hbm.at[pl.ds(base, _CHUNK)], idx_v)
                    pltpu.sync_copy(mref.at[pl.ds(base, _CHUNK)], rows_v)
                    pltpu.sync_copy(rows_v, acc.at[idx_v], add=True)
                    return carry

                lax.fori_loop(0, _NCHUNK, body, 0)

        if g == 0:
            @pl.when(c == 0)
            def _():
                def dbody(i, carry):
                    base = s * _EPW + i * _CHUNK
                    pltpu.sync_copy(dst_hbm.at[pl.ds(base, _CHUNK)], idx_v)
                    pltpu.sync_copy(ea_hbm.at[pl.ds(base, _CHUNK)], ea_v)
                    pltpu.sync_copy(ea_v, den_acc.at[idx_v], add=True)
                    return carry

                lax.fori_loop(0, _NCHUNK, dbody, 0)

        plsc.subcore_barrier()

        for gc in range(2):
            @pl.when(c == gc)
            def _():
                pltpu.sync_copy(acc.at[pl.ds(s * _NPW, _NPW)],
                                outs[2 * gc + g].at[pl.ds(s * _NPW, _NPW)])

        if g == 0:
            @pl.when(c == 0)
            def _():
                pltpu.sync_copy(den_acc.at[pl.ds(s * _NPW, _NPW)],
                                den_hbm.at[pl.ds(s * _NPW, _NPW)])

        plsc.subcore_barrier()


def _sc_scatter(m0, m1, m2, m3, dst, ea):
    z128 = jnp.zeros((_NPW, 128), jnp.float32)
    z8 = jnp.zeros((_NPW, 8), jnp.float32)
    mesh = plsc.VectorSubcoreMesh(core_axis_name="c", subcore_axis_name="s")
    f = functools.partial(
        pl.kernel,
        mesh=mesh,
        out_type=[jax.ShapeDtypeStruct((_NPAD, 128), jnp.float32)] * 4
        + [jax.ShapeDtypeStruct((_NPAD, 8), jnp.float32)],
        scratch_types=[
            pltpu.VMEM_SHARED((_NPAD, 128), jnp.float32),
            pltpu.VMEM_SHARED((_NPAD, 8), jnp.float32),
            pltpu.VMEM((_CHUNK,), jnp.int32),
            pltpu.VMEM((_CHUNK, 128), jnp.float32),
            pltpu.VMEM((_CHUNK, 8), jnp.float32),
        ],
    )(_sc_scatter_kernel)
    o0, o1, o2, o3, den_p = f(m0, m1, m2, m3, dst, ea, z128, z8)
    out_p = jnp.concatenate([o0, o1, o2, o3], axis=1)
    return out_p[:_N], den_p[:_N]


def kernel(x, edge_index, edge_attr, node_types, node_emb, edge_emb,
           Win, bin_, Wl, bl, Wr, br, We, att, bg, Wout, bout):
    src = edge_index[0]
    dst = edge_index[1]

    nt_oh = jax.nn.one_hot(node_types, 5, dtype=jnp.float32)
    h = _mm(x, Win, 1000) + bin_ + nt_oh @ node_emb

    attr_oh = jax.nn.one_hot(edge_attr, 8, dtype=jnp.float32)

    eye_h = jnp.eye(_HEADS, dtype=jnp.float32)
    expB = jnp.repeat(eye_h, _HID, axis=1)

    cnt = jnp.zeros((_N, 5), jnp.float32).at[dst, edge_attr].add(1.0)
    deg = cnt.sum(axis=1)
    emean = (cnt @ edge_emb) / jnp.maximum(deg, 1.0)[:, None]

    for l in range(_LAYERS):
        xl = _mm(h, Wl[l], 1000) + bl[l]
        xr = _mm(h, Wr[l], 1000) + br[l]
        ee_t = edge_emb @ We[l]
        ee_t8 = jnp.concatenate(
            [ee_t, jnp.zeros((3, _F), jnp.float32)], axis=0)
        ee_self = _mm(emean, We[l], 1000)
        attA = (att[l].reshape(_F, 1) * jnp.repeat(
            eye_h, _HID, axis=0)).astype(jnp.float32)

        xj = xl[src]
        xrd = xr[dst]
        m0, m1, m2, m3, ea = _edge_math(xj, xrd, attr_oh, ee_t8, attA, expB)
        msg = jnp.concatenate([m0, m1, m2, m3], axis=1)
        out_un = jnp.zeros((_N, _F), jnp.float32).at[dst].add(msg)
        den = jnp.zeros((_N, _HEADS), jnp.float32).at[dst].add(ea)

        us = xl + xr + ee_self
        us = jnp.maximum(us, _NEG * us).reshape(_N, _HEADS, _HID)
        eas = jnp.exp((us * att[l][None]).sum(-1))
        out_un = out_un + (xl.reshape(_N, _HEADS, _HID)
                           * eas[:, :, None]).reshape(_N, _F)
        den = den + eas

        out = out_un.reshape(_N, _HEADS, _HID) / (den + 1e-16)[:, :, None]
        h = jax.nn.relu(out.mean(axis=1) + bg[l])

    return _mm(h, Wout, 1000) + bout


# speedup vs baseline: 8.7593x; 1.2286x over previous
import functools

import jax
import jax.numpy as jnp
from jax import lax
from jax.experimental import pallas as pl
from jax.experimental.pallas import tpu as pltpu
from jax.experimental.pallas import tpu_sc as plsc

_N = 10000
_E = 160000
_HID = 64
_HEADS = 8
_LAYERS = 3
_NEG = 0.2
_F = _HEADS * _HID
_NSUB = 16
_EPW = _E // _NSUB
_CHUNK = 80
_NCHUNK = _EPW // _CHUNK
_NPAD = 10240
_NPW = _NPAD // _NSUB


def _mm_kernel(a_ref, b_ref, o_ref):
    o_ref[...] = jnp.dot(a_ref[...], b_ref[...],
                         preferred_element_type=jnp.float32)


def _mm(a, b, blk_rows):
    r, k = a.shape
    c = b.shape[1]
    return pl.pallas_call(
        _mm_kernel,
        grid=(r // blk_rows,),
        in_specs=[
            pl.BlockSpec((blk_rows, k), lambda i: (i, 0)),
            pl.BlockSpec((k, c), lambda i: (0, 0)),
        ],
        out_specs=pl.BlockSpec((blk_rows, c), lambda i: (i, 0)),
        out_shape=jax.ShapeDtypeStruct((r, c), jnp.float32),
    )(a, b)


_BE = 2000


def _edge_kernel(xj_ref, xrd_ref, oh_ref, eet_ref, attA_ref, expB_ref,
                 m0_ref, m1_ref, m2_ref, m3_ref, ea_ref):
    xj = xj_ref[...]
    u = xj + xrd_ref[...] + jnp.dot(oh_ref[...], eet_ref[...],
                                    preferred_element_type=jnp.float32)
    ul = jnp.maximum(u, _NEG * u)
    alpha = jnp.dot(ul, attA_ref[...], preferred_element_type=jnp.float32)
    ea = jnp.exp(alpha)
    m = xj * jnp.dot(ea, expB_ref[...],
                     preferred_element_type=jnp.float32)
    m0_ref[...] = m[:, 0:128]
    m1_ref[...] = m[:, 128:256]
    m2_ref[...] = m[:, 256:384]
    m3_ref[...] = m[:, 384:512]
    ea_ref[...] = ea


def _edge_math(xj, xrd, oh, ee_t8, attA, expB):
    grid = _E // _BE
    mspec = pl.BlockSpec((_BE, 128), lambda i: (i, 0))
    return pl.pallas_call(
        _edge_kernel,
        grid=(grid,),
        in_specs=[
            pl.BlockSpec((_BE, _F), lambda i: (i, 0)),
            pl.BlockSpec((_BE, _F), lambda i: (i, 0)),
            pl.BlockSpec((_BE, 8), lambda i: (i, 0)),
            pl.BlockSpec((8, _F), lambda i: (0, 0)),
            pl.BlockSpec((_F, 8), lambda i: (0, 0)),
            pl.BlockSpec((8, _F), lambda i: (0, 0)),
        ],
        out_specs=[mspec, mspec, mspec, mspec,
                   pl.BlockSpec((_BE, 8), lambda i: (i, 0))],
        out_shape=[jax.ShapeDtypeStruct((_E, 128), jnp.float32)] * 4
        + [jax.ShapeDtypeStruct((_E, 8), jnp.float32)],
    )(xj, xrd, oh, ee_t8, attA, expB)


_NWORK = 32
_EPWK = _E // _NWORK
_SCH = 40
_SNCH = _EPWK // _SCH


def _sc_scatter_kernel(m0, m1, m2, m3, dst_hbm, z128,
                       o0, o1, o2, o3,
                       acc, idx_v, rows_v):
    c = lax.axis_index("c")
    s = lax.axis_index("s")
    wid = s * 2 + c
    msgs = (m0, m1, m2, m3)
    outs = (o0, o1, o2, o3)

    for g in range(4):
        pltpu.sync_copy(z128, acc.at[pl.ds(s * _NPW, _NPW)])
        plsc.subcore_barrier()

        def body(i, carry):
            base = wid * _EPWK + i * _SCH
            pltpu.sync_copy(dst_hbm.at[pl.ds(base, _SCH)], idx_v)
            pltpu.sync_copy(msgs[g].at[pl.ds(base, _SCH)], rows_v)
            pltpu.sync_copy(rows_v, acc.at[idx_v], add=True)
            return carry

        lax.fori_loop(0, _SNCH, body, 0)
        plsc.subcore_barrier()

        pltpu.sync_copy(acc.at[pl.ds(s * _NPW, _NPW)],
                        outs[g].at[c, pl.ds(s * _NPW, _NPW)])
        plsc.subcore_barrier()


def _sc_scatter(m0, m1, m2, m3, dst):
    z128 = jnp.zeros((_NPW, 128), jnp.float32)
    mesh = plsc.VectorSubcoreMesh(core_axis_name="c", subcore_axis_name="s")
    f = functools.partial(
        pl.kernel,
        mesh=mesh,
        out_type=[jax.ShapeDtypeStruct((2, _NPAD, 128), jnp.float32)] * 4,
        scratch_types=[
            pltpu.VMEM_SHARED((_NPAD, 128), jnp.float32),
            pltpu.VMEM((_SCH,), jnp.int32),
            pltpu.VMEM((_SCH, 128), jnp.float32),
        ],
    )(_sc_scatter_kernel)
    o0, o1, o2, o3 = f(m0, m1, m2, m3, dst, z128)
    out_p = jnp.concatenate(
        [o[0] + o[1] for o in (o0, o1, o2, o3)], axis=1)
    return out_p[:_N]


def kernel(x, edge_index, edge_attr, node_types, node_emb, edge_emb,
           Win, bin_, Wl, bl, Wr, br, We, att, bg, Wout, bout):
    src = edge_index[0]
    dst = edge_index[1]

    nt_oh = jax.nn.one_hot(node_types, 5, dtype=jnp.float32)
    h = _mm(x, Win, 1000) + bin_ + nt_oh @ node_emb

    attr_oh = jax.nn.one_hot(edge_attr, 8, dtype=jnp.float32)

    eye_h = jnp.eye(_HEADS, dtype=jnp.float32)
    expB = jnp.repeat(eye_h, _HID, axis=1)

    cnt = jnp.zeros((_N, 5), jnp.float32).at[dst, edge_attr].add(1.0)
    deg = cnt.sum(axis=1)
    emean = (cnt @ edge_emb) / jnp.maximum(deg, 1.0)[:, None]

    for l in range(_LAYERS):
        xl = _mm(h, Wl[l], 1000) + bl[l]
        xr = _mm(h, Wr[l], 1000) + br[l]
        ee_t = edge_emb @ We[l]
        ee_t8 = jnp.concatenate(
            [ee_t, jnp.zeros((3, _F), jnp.float32)], axis=0)
        ee_self = _mm(emean, We[l], 1000)
        attA = (att[l].reshape(_F, 1) * jnp.repeat(
            eye_h, _HID, axis=0)).astype(jnp.float32)

        xj = xl[src]
        xrd = xr[dst]
        m0, m1, m2, m3, ea = _edge_math(xj, xrd, attr_oh, ee_t8, attA, expB)
        out_un = _sc_scatter(m0, m1, m2, m3, dst)
        den = jnp.zeros((_N, _HEADS), jnp.float32).at[dst].add(ea)

        us = xl + xr + ee_self
        us = jnp.maximum(us, _NEG * us).reshape(_N, _HEADS, _HID)
        eas = jnp.exp((us * att[l][None]).sum(-1))
        out_un = out_un + (xl.reshape(_N, _HEADS, _HID)
                           * eas[:, :, None]).reshape(_N, _F)
        den = den + eas

        out = out_un.reshape(_N, _HEADS, _HID) / (den + 1e-16)[:, :, None]
        h = jax.nn.relu(out.mean(axis=1) + bg[l])

    return _mm(h, Wout, 1000) + bout


# speedup vs baseline: 9.5200x; 1.0868x over previous
import functools

import jax
import jax.numpy as jnp
from jax import lax
from jax.experimental import pallas as pl
from jax.experimental.pallas import tpu as pltpu
from jax.experimental.pallas import tpu_sc as plsc

_N = 10000
_E = 160000
_HID = 64
_HEADS = 8
_LAYERS = 3
_NEG = 0.2
_F = _HEADS * _HID
_NSUB = 16
_EPW = _E // _NSUB
_CHUNK = 80
_NCHUNK = _EPW // _CHUNK
_NPAD = 10240
_NPW = _NPAD // _NSUB


def _mm_kernel(a_ref, b_ref, o_ref):
    o_ref[...] = jnp.dot(a_ref[...], b_ref[...],
                         preferred_element_type=jnp.float32)


def _mm(a, b, blk_rows):
    r, k = a.shape
    c = b.shape[1]
    return pl.pallas_call(
        _mm_kernel,
        grid=(r // blk_rows,),
        in_specs=[
            pl.BlockSpec((blk_rows, k), lambda i: (i, 0)),
            pl.BlockSpec((k, c), lambda i: (0, 0)),
        ],
        out_specs=pl.BlockSpec((blk_rows, c), lambda i: (i, 0)),
        out_shape=jax.ShapeDtypeStruct((r, c), jnp.float32),
    )(a, b)


_BE = 2000


def _edge_kernel(xj_ref, xrd_ref, oh_ref, eet_ref, attA_ref, expB_ref,
                 m0_ref, m1_ref, m2_ref, m3_ref, ea_ref):
    xj = xj_ref[...]
    u = xj + xrd_ref[...] + jnp.dot(oh_ref[...], eet_ref[...],
                                    preferred_element_type=jnp.float32)
    ul = jnp.maximum(u, _NEG * u)
    alpha = jnp.dot(ul, attA_ref[...], preferred_element_type=jnp.float32)
    ea = jnp.exp(alpha)
    m = xj * jnp.dot(ea, expB_ref[...],
                     preferred_element_type=jnp.float32)
    m0_ref[...] = m[:, 0:128]
    m1_ref[...] = m[:, 128:256]
    m2_ref[...] = m[:, 256:384]
    m3_ref[...] = m[:, 384:512]
    ea_ref[...] = ea


def _edge_math(xj, xrd, oh, ee_t8, attA, expB):
    grid = _E // _BE
    mspec = pl.BlockSpec((_BE, 128), lambda i: (i, 0))
    return pl.pallas_call(
        _edge_kernel,
        grid=(grid,),
        in_specs=[
            pl.BlockSpec((_BE, _F), lambda i: (i, 0)),
            pl.BlockSpec((_BE, _F), lambda i: (i, 0)),
            pl.BlockSpec((_BE, 8), lambda i: (i, 0)),
            pl.BlockSpec((8, _F), lambda i: (0, 0)),
            pl.BlockSpec((_F, 8), lambda i: (0, 0)),
            pl.BlockSpec((8, _F), lambda i: (0, 0)),
        ],
        out_specs=[mspec, mspec, mspec, mspec,
                   pl.BlockSpec((_BE, 8), lambda i: (i, 0))],
        out_shape=[jax.ShapeDtypeStruct((_E, 128), jnp.float32)] * 4
        + [jax.ShapeDtypeStruct((_E, 8), jnp.float32)],
    )(xj, xrd, oh, ee_t8, attA, expB)


_NWORK = 32
_EPWK = _E // _NWORK
_SCH = 40
_SNCH = _EPWK // _SCH


def _sc_scatter_kernel(m0, m1, m2, m3, dst_hbm, z128,
                       o0, o1, o2, o3,
                       acc, idx_v, rows_v):
    c = lax.axis_index("c")
    s = lax.axis_index("s")
    wid = s * 2 + c
    msgs = (m0, m1, m2, m3)
    outs = (o0, o1, o2, o3)

    for g in range(4):
        pltpu.sync_copy(z128, acc.at[pl.ds(s * _NPW, _NPW)])
        plsc.subcore_barrier()

        def body(i, carry):
            base = wid * _EPWK + i * _SCH
            pltpu.sync_copy(dst_hbm.at[pl.ds(base, _SCH)], idx_v)
            pltpu.sync_copy(msgs[g].at[pl.ds(base, _SCH)], rows_v)
            pltpu.sync_copy(rows_v, acc.at[idx_v], add=True)
            return carry

        lax.fori_loop(0, _SNCH, body, 0)
        plsc.subcore_barrier()

        pltpu.sync_copy(acc.at[pl.ds(s * _NPW, _NPW)],
                        outs[g].at[c, pl.ds(s * _NPW, _NPW)])
        plsc.subcore_barrier()


def _sc_gather_kernel(xl_hbm, xr_hbm, src_hbm, dst_hbm,
                      xj_hbm, xrd_hbm, idx_v, rows_v):
    c = lax.axis_index("c")
    s = lax.axis_index("s")
    wid = s * 2 + c

    for tab, idxs, out in ((xl_hbm, src_hbm, xj_hbm),
                           (xr_hbm, dst_hbm, xrd_hbm)):
        def body(i, carry, tab=tab, idxs=idxs, out=out):
            base = wid * _EPWK + i * _SCH
            pltpu.sync_copy(idxs.at[pl.ds(base, _SCH)], idx_v)
            pltpu.sync_copy(tab.at[idx_v], rows_v)
            pltpu.sync_copy(rows_v, out.at[pl.ds(base, _SCH)])
            return carry

        lax.fori_loop(0, _SNCH, body, 0)


def _sc_gather(xl, xr, src, dst):
    mesh = plsc.VectorSubcoreMesh(core_axis_name="c", subcore_axis_name="s")
    f = functools.partial(
        pl.kernel,
        mesh=mesh,
        out_type=[jax.ShapeDtypeStruct((_E, _F), jnp.float32)] * 2,
        scratch_types=[
            pltpu.VMEM((_SCH,), jnp.int32),
            pltpu.VMEM((_SCH, _F), jnp.float32),
        ],
    )(_sc_gather_kernel)
    return f(xl, xr, src, dst)


def _sc_scatter(m0, m1, m2, m3, dst):
    z128 = jnp.zeros((_NPW, 128), jnp.float32)
    mesh = plsc.VectorSubcoreMesh(core_axis_name="c", subcore_axis_name="s")
    f = functools.partial(
        pl.kernel,
        mesh=mesh,
        out_type=[jax.ShapeDtypeStruct((2, _NPAD, 128), jnp.float32)] * 4,
        scratch_types=[
            pltpu.VMEM_SHARED((_NPAD, 128), jnp.float32),
            pltpu.VMEM((_SCH,), jnp.int32),
            pltpu.VMEM((_SCH, 128), jnp.float32),
        ],
    )(_sc_scatter_kernel)
    o0, o1, o2, o3 = f(m0, m1, m2, m3, dst, z128)
    out_p = jnp.concatenate(
        [o[0] + o[1] for o in (o0, o1, o2, o3)], axis=1)
    return out_p[:_N]


def kernel(x, edge_index, edge_attr, node_types, node_emb, edge_emb,
           Win, bin_, Wl, bl, Wr, br, We, att, bg, Wout, bout):
    src = edge_index[0]
    dst = edge_index[1]

    nt_oh = jax.nn.one_hot(node_types, 5, dtype=jnp.float32)
    h = _mm(x, Win, 1000) + bin_ + nt_oh @ node_emb

    attr_oh = jax.nn.one_hot(edge_attr, 8, dtype=jnp.float32)

    eye_h = jnp.eye(_HEADS, dtype=jnp.float32)
    expB = jnp.repeat(eye_h, _HID, axis=1)

    cnt = jnp.zeros((_N, 5), jnp.float32).at[dst, edge_attr].add(1.0)
    deg = cnt.sum(axis=1)
    emean = (cnt @ edge_emb) / jnp.maximum(deg, 1.0)[:, None]

    for l in range(_LAYERS):
        xl = _mm(h, Wl[l], 1000) + bl[l]
        xr = _mm(h, Wr[l], 1000) + br[l]
        ee_t = edge_emb @ We[l]
        ee_t8 = jnp.concatenate(
            [ee_t, jnp.zeros((3, _F), jnp.float32)], axis=0)
        ee_self = _mm(emean, We[l], 1000)
        attA = (att[l].reshape(_F, 1) * jnp.repeat(
            eye_h, _HID, axis=0)).astype(jnp.float32)

        xj, xrd = _sc_gather(xl, xr, src, dst)
        m0, m1, m2, m3, ea = _edge_math(xj, xrd, attr_oh, ee_t8, attA, expB)
        out_un = _sc_scatter(m0, m1, m2, m3, dst)
        den = jnp.zeros((_N, _HEADS), jnp.float32).at[dst].add(ea)

        us = xl + xr + ee_self
        us = jnp.maximum(us, _NEG * us).reshape(_N, _HEADS, _HID)
        eas = jnp.exp((us * att[l][None]).sum(-1))
        out_un = out_un + (xl.reshape(_N, _HEADS, _HID)
                           * eas[:, :, None]).reshape(_N, _F)
        den = den + eas

        out = out_un.reshape(_N, _HEADS, _HID) / (den + 1e-16)[:, :, None]
        h = jax.nn.relu(out.mean(axis=1) + bg[l])

    return _mm(h, Wout, 1000) + bout
